# Initial kernel scaffold; baseline (speedup 1.0000x reference)
import jax
import jax.numpy as jnp
from jax.experimental import pallas as pl


def kernel(x, edge_index, edge_attr, W1, b1, W2, b2, gamma, beta):
    raise NotImplementedError("write your pallas kernel here")


# speedup vs baseline: 4.2614x; 4.2614x over previous
import functools

import jax
import jax.numpy as jnp
from jax import lax
from jax.experimental import pallas as pl
from jax.experimental.pallas import tpu as pltpu
from jax.experimental.pallas import tpu_sc as plsc

N_NODES = 10000
N_EDGES = 320000
D = 128

NC = 2
NS = 16
NW = NC * NS
E_PER_W = N_EDGES // NW
CH = 80
NCH = E_PER_W // CH
ROWS_PER_TILE = 624
TAIL_ROWS = N_NODES - NS * ROWS_PER_TILE


def _sc_segment_sum(edge_attr, jr, zeros_tile):
  mesh = plsc.VectorSubcoreMesh(core_axis_name="c", subcore_axis_name="s")

  @functools.partial(
      pl.kernel,
      out_type=jax.ShapeDtypeStruct((NC, N_NODES, D), jnp.float32),
      mesh=mesh,
      scratch_types=[
          pltpu.VMEM((NCH, CH), jnp.int32),
          pltpu.VMEM((CH, D), jnp.float32),
          pltpu.VMEM_SHARED((N_NODES, D), jnp.float32),
      ],
  )
  def k(ea_hbm, jr_hbm, z_hbm, out_hbm, idx_v, rows_v, acc):
    c = lax.axis_index("c")
    s = lax.axis_index("s")
    wid = c * NS + s

    pltpu.sync_copy(z_hbm, acc.at[pl.ds(s * ROWS_PER_TILE, ROWS_PER_TILE)])

    @pl.when(s == NS - 1)
    def _():
      pltpu.sync_copy(z_hbm.at[pl.ds(0, TAIL_ROWS)],
                      acc.at[pl.ds(NS * ROWS_PER_TILE, TAIL_ROWS)])

    pltpu.sync_copy(jr_hbm.at[wid], idx_v)
    plsc.subcore_barrier()

    base = wid * E_PER_W

    @pl.loop(0, NCH)
    def _(ch):
      pltpu.sync_copy(ea_hbm.at[pl.ds(base + ch * CH, CH)], rows_v)
      pltpu.sync_copy(rows_v, acc.at[idx_v.at[ch]], add=True)

    plsc.subcore_barrier()
    r0 = s * ROWS_PER_TILE
    pltpu.sync_copy(acc.at[pl.ds(r0, ROWS_PER_TILE)],
                    out_hbm.at[c, pl.ds(r0, ROWS_PER_TILE)])

    @pl.when(s == NS - 1)
    def _():
      t0 = NS * ROWS_PER_TILE
      pltpu.sync_copy(acc.at[pl.ds(t0, TAIL_ROWS)],
                      out_hbm.at[c, pl.ds(t0, TAIL_ROWS)])

  return k(edge_attr, jr, zeros_tile)


def _mlp_body(x_ref, p0_ref, p1_ref, w1x_ref, w1a_ref, b1_ref, w2_ref,
              b2_ref, g_ref, bt_ref, o_ref):
  x = x_ref[...]
  agg = p0_ref[...] + p1_ref[...]
  h = (jnp.dot(x, w1x_ref[...], preferred_element_type=jnp.float32)
       + jnp.dot(agg, w1a_ref[...], preferred_element_type=jnp.float32)
       + b1_ref[...])
  h = h * jax.nn.sigmoid(h)
  h = jnp.dot(h, w2_ref[...], preferred_element_type=jnp.float32) + b2_ref[...]
  mean = jnp.mean(h, axis=-1, keepdims=True)
  hc = h - mean
  var = jnp.mean(hc * hc, axis=-1, keepdims=True)
  o_ref[...] = x + hc * lax.rsqrt(var + 1e-5) * g_ref[...] + bt_ref[...]


def _tc_mlp(x, p0, p1, W1x, W1a, b1, W2, b2, gamma, beta):
  blk = 1000
  grid = N_NODES // blk
  row = lambda i: (i, 0)
  full = lambda i: (0, 0)
  return pl.pallas_call(
      _mlp_body,
      grid=(grid,),
      in_specs=[
          pl.BlockSpec((blk, D), row),
          pl.BlockSpec((blk, D), row),
          pl.BlockSpec((blk, D), row),
          pl.BlockSpec((D, 256), full),
          pl.BlockSpec((D, 256), full),
          pl.BlockSpec((1, 256), full),
          pl.BlockSpec((256, D), full),
          pl.BlockSpec((1, D), full),
          pl.BlockSpec((1, D), full),
          pl.BlockSpec((1, D), full),
      ],
      out_specs=pl.BlockSpec((blk, D), row),
      out_shape=jax.ShapeDtypeStruct((N_NODES, D), jnp.float32),
  )(x, p0, p1, W1x, W1a, b1, W2, b2, gamma, beta)


def kernel(x, edge_index, edge_attr, W1, b1, W2, b2, gamma, beta):
  j = edge_index[1].astype(jnp.int32)
  jr = j.reshape(NW, NCH, CH)
  zeros_tile = jnp.zeros((ROWS_PER_TILE, D), jnp.float32)
  parts = _sc_segment_sum(edge_attr, jr, zeros_tile)
  return _tc_mlp(x, parts[0], parts[1],
                 W1[:D], W1[D:], b1.reshape(1, -1),
                 W2, b2.reshape(1, -1),
                 gamma.reshape(1, -1), beta.reshape(1, -1))


# speedup vs baseline: 6.8701x; 1.6122x over previous
import functools

import jax
import jax.numpy as jnp
from jax import lax
from jax.experimental import pallas as pl
from jax.experimental.pallas import tpu as pltpu
from jax.experimental.pallas import tpu_sc as plsc

N_NODES = 10000
N_EDGES = 320000
D = 128

NC = 2
NS = 16
NW = NC * NS
E_PER_W = N_EDGES // NW
CH = 128
NCH = 78
TAIL_E = E_PER_W - NCH * CH
ROWS_PER_TILE = 624
TAIL_ROWS = N_NODES - NS * ROWS_PER_TILE


def _sc_segment_sum(edge_attr, jr_main, jr_tail, zeros_tile):
  mesh = plsc.VectorSubcoreMesh(core_axis_name="c", subcore_axis_name="s")

  @functools.partial(
      pl.kernel,
      out_type=jax.ShapeDtypeStruct((NC, N_NODES, D), jnp.float32),
      mesh=mesh,
      scratch_types=[
          pltpu.VMEM((NCH, CH), jnp.int32),
          pltpu.VMEM((TAIL_E,), jnp.int32),
          pltpu.VMEM((CH, D), jnp.float32),
          pltpu.VMEM((CH, D), jnp.float32),
          pltpu.VMEM_SHARED((N_NODES, D), jnp.float32),
          pltpu.SemaphoreType.DMA,
          pltpu.SemaphoreType.DMA,
      ],
  )
  def k(ea_hbm, jr_hbm, jt_hbm, z_hbm, out_hbm,
        idx_v, idx_t, rv0, rv1, acc, sem0, sem1):
    c = lax.axis_index("c")
    s = lax.axis_index("s")
    wid = c * NS + s
    base = wid * E_PER_W

    pltpu.sync_copy(z_hbm, acc.at[pl.ds(s * ROWS_PER_TILE, ROWS_PER_TILE)])

    @pl.when(s == NS - 1)
    def _():
      pltpu.sync_copy(z_hbm.at[pl.ds(0, TAIL_ROWS)],
                      acc.at[pl.ds(NS * ROWS_PER_TILE, TAIL_ROWS)])

    pltpu.sync_copy(jr_hbm.at[wid], idx_v)
    pltpu.sync_copy(jt_hbm.at[wid], idx_t)
    plsc.subcore_barrier()

    pltpu.async_copy(ea_hbm.at[pl.ds(base, CH)], rv0, sem0)
    pltpu.async_copy(ea_hbm.at[pl.ds(base + CH, CH)], rv1, sem1)

    @pl.loop(0, NCH, step=2)
    def _(ch):
      for b in range(2):
        rv = (rv0, rv1)[b]
        sem = (sem0, sem1)[b]
        cc = ch + b
        pltpu.make_async_copy(ea_hbm.at[pl.ds(base, CH)], rv, sem).wait()
        pltpu.sync_copy(rv, acc.at[idx_v.at[cc]], add=True)

        @pl.when(cc + 2 < NCH)
        def _():
          pltpu.async_copy(ea_hbm.at[pl.ds(base + (cc + 2) * CH, CH)], rv, sem)

    pltpu.sync_copy(ea_hbm.at[pl.ds(base + NCH * CH, TAIL_E)],
                    rv0.at[pl.ds(0, TAIL_E)])
    pltpu.sync_copy(rv0.at[pl.ds(0, TAIL_E)], acc.at[idx_t], add=True)

    plsc.subcore_barrier()
    r0 = s * ROWS_PER_TILE
    pltpu.sync_copy(acc.at[pl.ds(r0, ROWS_PER_TILE)],
                    out_hbm.at[c, pl.ds(r0, ROWS_PER_TILE)])

    @pl.when(s == NS - 1)
    def _():
      t0 = NS * ROWS_PER_TILE
      pltpu.sync_copy(acc.at[pl.ds(t0, TAIL_ROWS)],
                      out_hbm.at[c, pl.ds(t0, TAIL_ROWS)])

  return k(edge_attr, jr_main, jr_tail, zeros_tile)


def _mlp_body(x_ref, p0_ref, p1_ref, w1x_ref, w1a_ref, b1_ref, w2_ref,
              b2_ref, g_ref, bt_ref, o_ref):
  x = x_ref[...]
  agg = p0_ref[...] + p1_ref[...]
  h = (jnp.dot(x, w1x_ref[...], preferred_element_type=jnp.float32)
       + jnp.dot(agg, w1a_ref[...], preferred_element_type=jnp.float32)
       + b1_ref[...])
  h = h * jax.nn.sigmoid(h)
  h = jnp.dot(h, w2_ref[...], preferred_element_type=jnp.float32) + b2_ref[...]
  mean = jnp.mean(h, axis=-1, keepdims=True)
  hc = h - mean
  var = jnp.mean(hc * hc, axis=-1, keepdims=True)
  o_ref[...] = x + hc * lax.rsqrt(var + 1e-5) * g_ref[...] + bt_ref[...]


def _tc_mlp(x, p0, p1, W1x, W1a, b1, W2, b2, gamma, beta):
  blk = 1000
  grid = N_NODES // blk
  row = lambda i: (i, 0)
  full = lambda i: (0, 0)
  return pl.pallas_call(
      _mlp_body,
      grid=(grid,),
      in_specs=[
          pl.BlockSpec((blk, D), row),
          pl.BlockSpec((blk, D), row),
          pl.BlockSpec((blk, D), row),
          pl.BlockSpec((D, 256), full),
          pl.BlockSpec((D, 256), full),
          pl.BlockSpec((1, 256), full),
          pl.BlockSpec((256, D), full),
          pl.BlockSpec((1, D), full),
          pl.BlockSpec((1, D), full),
          pl.BlockSpec((1, D), full),
      ],
      out_specs=pl.BlockSpec((blk, D), row),
      out_shape=jax.ShapeDtypeStruct((N_NODES, D), jnp.float32),
  )(x, p0, p1, W1x, W1a, b1, W2, b2, gamma, beta)


def kernel(x, edge_index, edge_attr, W1, b1, W2, b2, gamma, beta):
  j = edge_index[1].astype(jnp.int32).reshape(NW, E_PER_W)
  jr_main = j[:, :NCH * CH].reshape(NW, NCH, CH)
  jr_tail = j[:, NCH * CH:]
  zeros_tile = jnp.zeros((ROWS_PER_TILE, D), jnp.float32)
  parts = _sc_segment_sum(edge_attr, jr_main, jr_tail, zeros_tile)
  return _tc_mlp(x, parts[0], parts[1],
                 W1[:D], W1[D:], b1.reshape(1, -1),
                 W2, b2.reshape(1, -1),
                 gamma.reshape(1, -1), beta.reshape(1, -1))


# speedup vs baseline: 7.3702x; 1.0728x over previous
import functools

import jax
import jax.numpy as jnp
from jax import lax
from jax.experimental import pallas as pl
from jax.experimental.pallas import tpu as pltpu
from jax.experimental.pallas import tpu_sc as plsc

N_NODES = 10000
N_EDGES = 320000
D = 128

NC = 2
NS = 16
NW = NC * NS
E_PER_W = N_EDGES // NW
CH = 128
NCH = 78
TAIL_E = E_PER_W - NCH * CH
ROWS_PER_TILE = 624
TAIL_ROWS = N_NODES - NS * ROWS_PER_TILE


def _sc_segment_sum(j32, edge_attr, zeros_tile):
  mesh = plsc.VectorSubcoreMesh(core_axis_name="c", subcore_axis_name="s")

  @functools.partial(
      pl.kernel,
      out_type=jax.ShapeDtypeStruct((NC, N_NODES, D), jnp.float32),
      mesh=mesh,
      scratch_types=[
          pltpu.VMEM((NCH, CH), jnp.int32),
          pltpu.VMEM((TAIL_E,), jnp.int32),
          pltpu.VMEM((CH, D), jnp.float32),
          pltpu.VMEM((CH, D), jnp.float32),
          pltpu.VMEM_SHARED((N_NODES, D), jnp.float32),
          pltpu.SemaphoreType.DMA,
          pltpu.SemaphoreType.DMA,
          pltpu.SemaphoreType.DMA,
      ],
  )
  def k(j_hbm, ea_hbm, z_hbm, out_hbm,
        idx_v, idx_t, rv0, rv1, acc, sem0, sem1, semi):
    c = lax.axis_index("c")
    s = lax.axis_index("s")
    wid = c * NS + s
    base = wid * E_PER_W

    for cc in range(NCH):
      pltpu.async_copy(j_hbm.at[pl.ds(base + cc * CH, CH)],
                       idx_v.at[cc], semi)
    pltpu.async_copy(j_hbm.at[pl.ds(base + NCH * CH, TAIL_E)], idx_t, semi)

    pltpu.sync_copy(z_hbm, acc.at[pl.ds(s * ROWS_PER_TILE, ROWS_PER_TILE)])

    @pl.when(s == NS - 1)
    def _():
      pltpu.sync_copy(z_hbm.at[pl.ds(0, TAIL_ROWS)],
                      acc.at[pl.ds(NS * ROWS_PER_TILE, TAIL_ROWS)])

    for cc in range(NCH):
      pltpu.make_async_copy(j_hbm.at[pl.ds(base, CH)],
                            idx_v.at[cc], semi).wait()
    pltpu.make_async_copy(j_hbm.at[pl.ds(base, TAIL_E)], idx_t, semi).wait()
    plsc.subcore_barrier()

    pltpu.async_copy(ea_hbm.at[pl.ds(base, CH)], rv0, sem0)
    pltpu.async_copy(ea_hbm.at[pl.ds(base + CH, CH)], rv1, sem1)

    @pl.loop(0, NCH, step=2)
    def _(ch):
      for b in range(2):
        rv = (rv0, rv1)[b]
        sem = (sem0, sem1)[b]
        cc = ch + b
        pltpu.make_async_copy(ea_hbm.at[pl.ds(base, CH)], rv, sem).wait()
        pltpu.sync_copy(rv, acc.at[idx_v.at[cc]], add=True)

        @pl.when(cc + 2 < NCH)
        def _():
          pltpu.async_copy(ea_hbm.at[pl.ds(base + (cc + 2) * CH, CH)], rv, sem)

    pltpu.sync_copy(ea_hbm.at[pl.ds(base + NCH * CH, TAIL_E)],
                    rv0.at[pl.ds(0, TAIL_E)])
    pltpu.sync_copy(rv0.at[pl.ds(0, TAIL_E)], acc.at[idx_t], add=True)

    plsc.subcore_barrier()
    r0 = s * ROWS_PER_TILE
    pltpu.sync_copy(acc.at[pl.ds(r0, ROWS_PER_TILE)],
                    out_hbm.at[c, pl.ds(r0, ROWS_PER_TILE)])

    @pl.when(s == NS - 1)
    def _():
      t0 = NS * ROWS_PER_TILE
      pltpu.sync_copy(acc.at[pl.ds(t0, TAIL_ROWS)],
                      out_hbm.at[c, pl.ds(t0, TAIL_ROWS)])

  return k(j32, edge_attr, zeros_tile)


def _mlp_body(x_ref, pa_ref, pb_ref, w1_ref, b1_ref, w2_ref,
              b2_ref, g_ref, bt_ref, o_ref):
  x = x_ref[...]
  agg = pa_ref[0] + pb_ref[0]
  xin = jnp.concatenate([x, agg], axis=-1)
  h = jnp.dot(xin, w1_ref[...], preferred_element_type=jnp.float32) + b1_ref[...]
  h = h * jax.nn.sigmoid(h)
  h = jnp.dot(h, w2_ref[...], preferred_element_type=jnp.float32) + b2_ref[...]
  mean = jnp.mean(h, axis=-1, keepdims=True)
  hc = h - mean
  var = jnp.mean(hc * hc, axis=-1, keepdims=True)
  o_ref[...] = x + hc * lax.rsqrt(var + 1e-5) * g_ref[...] + bt_ref[...]


def _tc_mlp(x, parts, W1, b1, W2, b2, gamma, beta):
  blk = 1000
  grid = N_NODES // blk
  row = lambda i: (i, 0)
  full = lambda i: (0, 0)
  return pl.pallas_call(
      _mlp_body,
      grid=(grid,),
      in_specs=[
          pl.BlockSpec((blk, D), row),
          pl.BlockSpec((1, blk, D), lambda i: (0, i, 0)),
          pl.BlockSpec((1, blk, D), lambda i: (1, i, 0)),
          pl.BlockSpec((256, 256), full),
          pl.BlockSpec((1, 256), full),
          pl.BlockSpec((256, D), full),
          pl.BlockSpec((1, D), full),
          pl.BlockSpec((1, D), full),
          pl.BlockSpec((1, D), full),
      ],
      out_specs=pl.BlockSpec((blk, D), row),
      out_shape=jax.ShapeDtypeStruct((N_NODES, D), jnp.float32),
  )(x, parts, parts, W1, b1, W2, b2, gamma, beta)


def kernel(x, edge_index, edge_attr, W1, b1, W2, b2, gamma, beta):
  j32 = edge_index[1].astype(jnp.int32)
  zeros_tile = jnp.zeros((ROWS_PER_TILE, D), jnp.float32)
  parts = _sc_segment_sum(j32, edge_attr, zeros_tile)
  return _tc_mlp(x, parts, W1, b1.reshape(1, -1),
                 W2, b2.reshape(1, -1),
                 gamma.reshape(1, -1), beta.reshape(1, -1))


# speedup vs baseline: 7.9000x; 1.0719x over previous
import functools

import jax
import jax.numpy as jnp
from jax import lax
from jax.experimental import pallas as pl
from jax.experimental.pallas import tpu as pltpu
from jax.experimental.pallas import tpu_sc as plsc

N_NODES = 10000
N_EDGES = 320000
D = 128

NC = 2
NS = 16
NW = NC * NS
CH = 128
NCH = 78
E_PER_W = NCH * CH
X_BASE = NW * E_PER_W
NX = (N_EDGES - X_BASE) // CH
JROWS = N_EDGES // CH
ROWS_PER_TILE = 624
TAIL_ROWS = N_NODES - NS * ROWS_PER_TILE


def _sc_segment_sum(jd, edge_attr, zeros_tile):
  mesh = plsc.VectorSubcoreMesh(core_axis_name="c", subcore_axis_name="s")

  @functools.partial(
      pl.kernel,
      out_type=jax.ShapeDtypeStruct((NC, N_NODES, D), jnp.float32),
      mesh=mesh,
      scratch_types=[
          pltpu.VMEM((88, CH), jnp.int32),
          pltpu.VMEM((8, CH), jnp.int32),
          pltpu.VMEM((CH, D), jnp.float32),
          pltpu.VMEM((CH, D), jnp.float32),
          pltpu.VMEM_SHARED((N_NODES, D), jnp.float32),
          pltpu.SemaphoreType.DMA,
          pltpu.SemaphoreType.DMA,
          pltpu.SemaphoreType.DMA,
      ],
  )
  def k(jd_hbm, ea_hbm, z_hbm, out_hbm,
        idx_v, idx_x, rv0, rv1, acc, sem0, sem1, semi):
    c = lax.axis_index("c")
    s = lax.axis_index("s")
    wid = c * NS + s
    base = wid * E_PER_W

    row0 = wid * NCH
    r0 = pl.multiple_of((row0 >> 3) << 3, 8)
    o = row0 - r0
    pltpu.async_copy(jd_hbm.at[pl.ds(r0, 88)], idx_v, semi)

    @pl.when(wid < NX)
    def _():
      pltpu.async_copy(jd_hbm.at[pl.ds(NW * NCH, 8)], idx_x, sem1)

    pltpu.sync_copy(z_hbm, acc.at[pl.ds(s * ROWS_PER_TILE, ROWS_PER_TILE)])

    @pl.when(s == NS - 1)
    def _():
      pltpu.sync_copy(z_hbm.at[pl.ds(0, TAIL_ROWS)],
                      acc.at[pl.ds(NS * ROWS_PER_TILE, TAIL_ROWS)])

    pltpu.make_async_copy(jd_hbm.at[pl.ds(r0, 88)], idx_v, semi).wait()

    @pl.when(wid < NX)
    def _():
      pltpu.make_async_copy(jd_hbm.at[pl.ds(NW * NCH, 8)], idx_x, sem1).wait()

    plsc.subcore_barrier()

    pltpu.async_copy(ea_hbm.at[pl.ds(base, CH)], rv0, sem0)
    pltpu.async_copy(ea_hbm.at[pl.ds(base + CH, CH)], rv1, sem1)

    @pl.loop(0, NCH, step=2)
    def _(ch):
      for b in range(2):
        rv = (rv0, rv1)[b]
        sem = (sem0, sem1)[b]
        cc = ch + b
        pltpu.make_async_copy(ea_hbm.at[pl.ds(base, CH)], rv, sem).wait()
        pltpu.sync_copy(rv, acc.at[idx_v.at[o + cc]], add=True)

        @pl.when(cc + 2 < NCH)
        def _():
          pltpu.async_copy(ea_hbm.at[pl.ds(base + (cc + 2) * CH, CH)], rv, sem)

    @pl.when(wid < NX)
    def _():
      pltpu.sync_copy(ea_hbm.at[pl.ds(X_BASE + wid * CH, CH)], rv0)
      pltpu.sync_copy(rv0, acc.at[idx_x.at[wid]], add=True)

    plsc.subcore_barrier()
    r0 = s * ROWS_PER_TILE
    pltpu.sync_copy(acc.at[pl.ds(r0, ROWS_PER_TILE)],
                    out_hbm.at[c, pl.ds(r0, ROWS_PER_TILE)])

    @pl.when(s == NS - 1)
    def _():
      t0 = NS * ROWS_PER_TILE
      pltpu.sync_copy(acc.at[pl.ds(t0, TAIL_ROWS)],
                      out_hbm.at[c, pl.ds(t0, TAIL_ROWS)])

  return k(jd, edge_attr, zeros_tile)


def _mlp_body(x_ref, pa_ref, pb_ref, w1_ref, b1_ref, w2_ref,
              b2_ref, g_ref, bt_ref, o_ref):
  x = x_ref[...]
  agg = pa_ref[0] + pb_ref[0]
  xin = jnp.concatenate([x, agg], axis=-1)
  h = jnp.dot(xin, w1_ref[...], preferred_element_type=jnp.float32) + b1_ref[...]
  h = h * jax.nn.sigmoid(h)
  h = jnp.dot(h, w2_ref[...], preferred_element_type=jnp.float32) + b2_ref[...]
  mean = jnp.mean(h, axis=-1, keepdims=True)
  hc = h - mean
  var = jnp.mean(hc * hc, axis=-1, keepdims=True)
  o_ref[...] = x + hc * lax.rsqrt(var + 1e-5) * g_ref[...] + bt_ref[...]


def _tc_mlp(x, parts, W1, b1, W2, b2, gamma, beta):
  blk = 1000
  grid = N_NODES // blk
  row = lambda i: (i, 0)
  full = lambda i: (0, 0)
  return pl.pallas_call(
      _mlp_body,
      grid=(grid,),
      in_specs=[
          pl.BlockSpec((blk, D), row),
          pl.BlockSpec((1, blk, D), lambda i: (0, i, 0)),
          pl.BlockSpec((1, blk, D), lambda i: (1, i, 0)),
          pl.BlockSpec((256, 256), full),
          pl.BlockSpec((1, 256), full),
          pl.BlockSpec((256, D), full),
          pl.BlockSpec((1, D), full),
          pl.BlockSpec((1, D), full),
          pl.BlockSpec((1, D), full),
      ],
      out_specs=pl.BlockSpec((blk, D), row),
      out_shape=jax.ShapeDtypeStruct((N_NODES, D), jnp.float32),
  )(x, parts, parts, W1, b1, W2, b2, gamma, beta)


def kernel(x, edge_index, edge_attr, W1, b1, W2, b2, gamma, beta):
  jd = edge_index.astype(jnp.int32).reshape(2, JROWS, CH)[1]
  jd = jnp.concatenate([jd, jnp.zeros((4, CH), jnp.int32)], axis=0)
  zeros_tile = jnp.zeros((ROWS_PER_TILE, D), jnp.float32)
  parts = _sc_segment_sum(jd, edge_attr, zeros_tile)
  return _tc_mlp(x, parts, W1, b1.reshape(1, -1),
                 W2, b2.reshape(1, -1),
                 gamma.reshape(1, -1), beta.reshape(1, -1))


# speedup vs baseline: 8.1046x; 1.0259x over previous
import functools

import jax
import jax.numpy as jnp
from jax import lax
from jax.experimental import pallas as pl
from jax.experimental.pallas import tpu as pltpu
from jax.experimental.pallas import tpu_sc as plsc

N_NODES = 10000
N_EDGES = 320000
D = 128

NC = 2
NS = 16
NW = NC * NS
CH = 128
NCH = 78
E_PER_W = NCH * CH
X_BASE = NW * E_PER_W
NX = (N_EDGES - X_BASE) // CH
JROWS = N_EDGES // CH
ROWS_PER_TILE = 624
TAIL_ROWS = N_NODES - NS * ROWS_PER_TILE


def _sc_segment_sum(jd, edge_attr, zeros_tile):
  mesh = plsc.VectorSubcoreMesh(core_axis_name="c", subcore_axis_name="s")

  @functools.partial(
      pl.kernel,
      out_type=jax.ShapeDtypeStruct((NC, N_NODES, D), jnp.float32),
      mesh=mesh,
      scratch_types=[
          pltpu.VMEM((88, CH), jnp.int32),
          pltpu.VMEM((8, CH), jnp.int32),
          pltpu.VMEM((CH, D), jnp.float32),
          pltpu.VMEM((CH, D), jnp.float32),
          pltpu.VMEM_SHARED((N_NODES, D), jnp.float32),
          pltpu.SemaphoreType.DMA,
          pltpu.SemaphoreType.DMA,
          pltpu.SemaphoreType.DMA,
      ],
  )
  def k(jd_hbm, ea_hbm, z_hbm, out_hbm,
        idx_v, idx_x, rv0, rv1, acc, sem0, sem1, semi):
    c = lax.axis_index("c")
    s = lax.axis_index("s")
    wid = c * NS + s
    base = wid * E_PER_W

    row0 = wid * NCH
    r0 = pl.multiple_of((row0 >> 3) << 3, 8)
    o = row0 - r0
    xtra = (wid & 7) == 0
    t = wid >> 3
    pltpu.async_copy(jd_hbm.at[pl.ds(r0, 88)], idx_v, semi)

    @pl.when(xtra)
    def _():
      pltpu.async_copy(jd_hbm.at[pl.ds(NW * NCH, 8)], idx_x, sem1)

    pltpu.sync_copy(z_hbm, acc.at[pl.ds(s * ROWS_PER_TILE, ROWS_PER_TILE)])

    @pl.when(s == NS - 1)
    def _():
      pltpu.sync_copy(z_hbm.at[pl.ds(0, TAIL_ROWS)],
                      acc.at[pl.ds(NS * ROWS_PER_TILE, TAIL_ROWS)])

    pltpu.make_async_copy(jd_hbm.at[pl.ds(r0, 88)], idx_v, semi).wait()

    @pl.when(xtra)
    def _():
      pltpu.make_async_copy(jd_hbm.at[pl.ds(NW * NCH, 8)], idx_x, sem1).wait()

    plsc.subcore_barrier()

    pltpu.async_copy(ea_hbm.at[pl.ds(base, CH)], rv0, sem0)
    pltpu.async_copy(ea_hbm.at[pl.ds(base + CH, CH)], rv1, sem1)

    @pl.loop(0, NCH, step=2)
    def _(ch):
      for b in range(2):
        rv = (rv0, rv1)[b]
        sem = (sem0, sem1)[b]
        cc = ch + b
        pltpu.make_async_copy(ea_hbm.at[pl.ds(base, CH)], rv, sem).wait()
        pltpu.sync_copy(rv, acc.at[idx_v.at[o + cc]], add=True)

        @pl.when(cc + 2 < NCH)
        def _():
          pltpu.async_copy(ea_hbm.at[pl.ds(base + (cc + 2) * CH, CH)], rv, sem)

    @pl.when(xtra)
    def _():
      pltpu.sync_copy(ea_hbm.at[pl.ds(X_BASE + t * CH, CH)], rv0)
      pltpu.sync_copy(rv0, acc.at[idx_x.at[t]], add=True)

    plsc.subcore_barrier()
    r0 = s * ROWS_PER_TILE
    pltpu.sync_copy(acc.at[pl.ds(r0, ROWS_PER_TILE)],
                    out_hbm.at[c, pl.ds(r0, ROWS_PER_TILE)])

    @pl.when(s == NS - 1)
    def _():
      t0 = NS * ROWS_PER_TILE
      pltpu.sync_copy(acc.at[pl.ds(t0, TAIL_ROWS)],
                      out_hbm.at[c, pl.ds(t0, TAIL_ROWS)])

  return k(jd, edge_attr, zeros_tile)


def _mlp_body(x_ref, pa_ref, pb_ref, w1_ref, b1_ref, w2_ref,
              b2_ref, g_ref, bt_ref, o_ref):
  x = x_ref[...]
  agg = pa_ref[0] + pb_ref[0]
  xin = jnp.concatenate([x, agg], axis=-1)
  h = jnp.dot(xin, w1_ref[...], preferred_element_type=jnp.float32) + b1_ref[...]
  h = h * jax.nn.sigmoid(h)
  h = jnp.dot(h, w2_ref[...], preferred_element_type=jnp.float32) + b2_ref[...]
  mean = jnp.mean(h, axis=-1, keepdims=True)
  hc = h - mean
  var = jnp.mean(hc * hc, axis=-1, keepdims=True)
  o_ref[...] = x + hc * lax.rsqrt(var + 1e-5) * g_ref[...] + bt_ref[...]


def _tc_mlp(x, parts, W1, b1, W2, b2, gamma, beta):
  blk = 2000
  grid = N_NODES // blk
  row = lambda i: (i, 0)
  full = lambda i: (0, 0)
  return pl.pallas_call(
      _mlp_body,
      grid=(grid,),
      in_specs=[
          pl.BlockSpec((blk, D), row),
          pl.BlockSpec((1, blk, D), lambda i: (0, i, 0)),
          pl.BlockSpec((1, blk, D), lambda i: (1, i, 0)),
          pl.BlockSpec((256, 256), full),
          pl.BlockSpec((1, 256), full),
          pl.BlockSpec((256, D), full),
          pl.BlockSpec((1, D), full),
          pl.BlockSpec((1, D), full),
          pl.BlockSpec((1, D), full),
      ],
      out_specs=pl.BlockSpec((blk, D), row),
      out_shape=jax.ShapeDtypeStruct((N_NODES, D), jnp.float32),
  )(x, parts, parts, W1, b1, W2, b2, gamma, beta)


def kernel(x, edge_index, edge_attr, W1, b1, W2, b2, gamma, beta):
  jd = edge_index.astype(jnp.int32).reshape(2, JROWS, CH)[1]
  jd = jnp.concatenate([jd, jnp.zeros((4, CH), jnp.int32)], axis=0)
  zeros_tile = jnp.zeros((ROWS_PER_TILE, D), jnp.float32)
  parts = _sc_segment_sum(jd, edge_attr, zeros_tile)
  return _tc_mlp(x, parts, W1, b1.reshape(1, -1),
                 W2, b2.reshape(1, -1),
                 gamma.reshape(1, -1), beta.reshape(1, -1))


# speedup vs baseline: 8.2066x; 1.0126x over previous
import functools

import jax
import jax.numpy as jnp
from jax import lax
from jax.experimental import pallas as pl
from jax.experimental.pallas import tpu as pltpu
from jax.experimental.pallas import tpu_sc as plsc

N_NODES = 10000
N_EDGES = 320000
D = 128

NC = 2
NS = 16
NW = NC * NS
CH = 128
NCH = 78
E_PER_W = NCH * CH
X_BASE = NW * E_PER_W
NX = (N_EDGES - X_BASE) // CH
JROWS = N_EDGES // CH
ROWS_PER_TILE = 624
TAIL_ROWS = N_NODES - NS * ROWS_PER_TILE


def _sc_segment_sum(jd, edge_attr, zeros_tile):
  mesh = plsc.VectorSubcoreMesh(core_axis_name="c", subcore_axis_name="s")

  @functools.partial(
      pl.kernel,
      out_type=jax.ShapeDtypeStruct((NC, N_NODES, D), jnp.float32),
      mesh=mesh,
      scratch_types=[
          pltpu.VMEM((88, CH), jnp.int32),
          pltpu.VMEM((8, CH), jnp.int32),
          pltpu.VMEM((CH, D), jnp.float32),
          pltpu.VMEM((CH, D), jnp.float32),
          pltpu.VMEM_SHARED((N_NODES, D), jnp.float32),
          pltpu.SemaphoreType.DMA,
          pltpu.SemaphoreType.DMA,
          pltpu.SemaphoreType.DMA,
          pltpu.SemaphoreType.DMA,
      ],
  )
  def k(jd_hbm, ea_hbm, z_hbm, out_hbm,
        idx_v, idx_x, rv0, rv1, acc, sem0, sem1, semi, semx):
    c = lax.axis_index("c")
    s = lax.axis_index("s")
    wid = c * NS + s
    base = wid * E_PER_W

    row0 = wid * NCH
    r0 = pl.multiple_of((row0 >> 3) << 3, 8)
    o = row0 - r0
    xtra = (wid & 7) == 0
    t = wid >> 3
    pltpu.async_copy(jd_hbm.at[pl.ds(r0, 88)], idx_v, semi)

    @pl.when(xtra)
    def _():
      pltpu.async_copy(jd_hbm.at[pl.ds(NW * NCH, 8)], idx_x, semx)

    pltpu.async_copy(ea_hbm.at[pl.ds(base, CH)], rv0, sem0)
    pltpu.async_copy(ea_hbm.at[pl.ds(base + CH, CH)], rv1, sem1)

    pltpu.sync_copy(z_hbm, acc.at[pl.ds(s * ROWS_PER_TILE, ROWS_PER_TILE)])

    @pl.when(s == NS - 1)
    def _():
      pltpu.sync_copy(z_hbm.at[pl.ds(0, TAIL_ROWS)],
                      acc.at[pl.ds(NS * ROWS_PER_TILE, TAIL_ROWS)])

    pltpu.make_async_copy(jd_hbm.at[pl.ds(r0, 88)], idx_v, semi).wait()

    @pl.when(xtra)
    def _():
      pltpu.make_async_copy(jd_hbm.at[pl.ds(NW * NCH, 8)], idx_x, semx).wait()

    plsc.subcore_barrier()

    @pl.loop(0, NCH, step=2)
    def _(ch):
      for b in range(2):
        rv = (rv0, rv1)[b]
        sem = (sem0, sem1)[b]
        cc = ch + b
        pltpu.make_async_copy(ea_hbm.at[pl.ds(base, CH)], rv, sem).wait()
        pltpu.sync_copy(rv, acc.at[idx_v.at[o + cc]], add=True)

        @pl.when(cc + 2 < NCH)
        def _():
          pltpu.async_copy(ea_hbm.at[pl.ds(base + (cc + 2) * CH, CH)], rv, sem)

    @pl.when(xtra)
    def _():
      pltpu.sync_copy(ea_hbm.at[pl.ds(X_BASE + t * CH, CH)], rv0)
      pltpu.sync_copy(rv0, acc.at[idx_x.at[t]], add=True)

    plsc.subcore_barrier()
    r0 = s * ROWS_PER_TILE
    pltpu.sync_copy(acc.at[pl.ds(r0, ROWS_PER_TILE)],
                    out_hbm.at[c, pl.ds(r0, ROWS_PER_TILE)])

    @pl.when(s == NS - 1)
    def _():
      t0 = NS * ROWS_PER_TILE
      pltpu.sync_copy(acc.at[pl.ds(t0, TAIL_ROWS)],
                      out_hbm.at[c, pl.ds(t0, TAIL_ROWS)])

  return k(jd, edge_attr, zeros_tile)


def _mlp_body(x_ref, pa_ref, pb_ref, w1_ref, b1_ref, w2_ref,
              b2_ref, g_ref, bt_ref, o_ref):
  x = x_ref[...]
  agg = pa_ref[0] + pb_ref[0]
  xin = jnp.concatenate([x, agg], axis=-1)
  h = jnp.dot(xin, w1_ref[...], preferred_element_type=jnp.float32) + b1_ref[...]
  h = h * jax.nn.sigmoid(h)
  h = jnp.dot(h, w2_ref[...], preferred_element_type=jnp.float32) + b2_ref[...]
  mean = jnp.mean(h, axis=-1, keepdims=True)
  hc = h - mean
  var = jnp.mean(hc * hc, axis=-1, keepdims=True)
  o_ref[...] = x + hc * lax.rsqrt(var + 1e-5) * g_ref[...] + bt_ref[...]


def _tc_mlp(x, parts, W1, b1, W2, b2, gamma, beta):
  blk = 5000
  grid = N_NODES // blk
  row = lambda i: (i, 0)
  full = lambda i: (0, 0)
  return pl.pallas_call(
      _mlp_body,
      grid=(grid,),
      in_specs=[
          pl.BlockSpec((blk, D), row),
          pl.BlockSpec((1, blk, D), lambda i: (0, i, 0)),
          pl.BlockSpec((1, blk, D), lambda i: (1, i, 0)),
          pl.BlockSpec((256, 256), full),
          pl.BlockSpec((1, 256), full),
          pl.BlockSpec((256, D), full),
          pl.BlockSpec((1, D), full),
          pl.BlockSpec((1, D), full),
          pl.BlockSpec((1, D), full),
      ],
      out_specs=pl.BlockSpec((blk, D), row),
      out_shape=jax.ShapeDtypeStruct((N_NODES, D), jnp.float32),
  )(x, parts, parts, W1, b1, W2, b2, gamma, beta)


def kernel(x, edge_index, edge_attr, W1, b1, W2, b2, gamma, beta):
  jd = edge_index.astype(jnp.int32).reshape(2, JROWS, CH)[1]
  jd = jnp.concatenate([jd, jnp.zeros((4, CH), jnp.int32)], axis=0)
  zeros_tile = jnp.zeros((ROWS_PER_TILE, D), jnp.float32)
  parts = _sc_segment_sum(jd, edge_attr, zeros_tile)
  return _tc_mlp(x, parts, W1, b1.reshape(1, -1),
                 W2, b2.reshape(1, -1),
                 gamma.reshape(1, -1), beta.reshape(1, -1))


# speedup vs baseline: 8.3742x; 1.0204x over previous
import functools

import jax
import jax.numpy as jnp
from jax import lax
from jax.experimental import pallas as pl
from jax.experimental.pallas import tpu as pltpu
from jax.experimental.pallas import tpu_sc as plsc

N_NODES = 10000
N_EDGES = 320000
D = 128

NC = 2
NS = 16
NW = NC * NS
CH = 128
NCH = 78
E_PER_W = NCH * CH
X_BASE = NW * E_PER_W
NX = (N_EDGES - X_BASE) // CH
JROWS = N_EDGES // CH
ROWS_PER_TILE = 624
TAIL_ROWS = N_NODES - NS * ROWS_PER_TILE


def _sc_segment_sum(edge_index, edge_attr, zeros_tile):
  mesh = plsc.VectorSubcoreMesh(core_axis_name="c", subcore_axis_name="s")

  @functools.partial(
      pl.kernel,
      out_type=jax.ShapeDtypeStruct((NC, N_NODES, D), jnp.float32),
      mesh=mesh,
      scratch_types=[
          pltpu.VMEM((2, CH), jnp.int32),
          pltpu.VMEM((2, CH), jnp.int32),
          pltpu.VMEM((2, CH), jnp.int32),
          pltpu.VMEM((CH, D), jnp.float32),
          pltpu.VMEM((CH, D), jnp.float32),
          pltpu.VMEM_SHARED((N_NODES, D), jnp.float32),
          pltpu.SemaphoreType.DMA,
          pltpu.SemaphoreType.DMA,
          pltpu.SemaphoreType.DMA,
          pltpu.SemaphoreType.DMA,
          pltpu.SemaphoreType.DMA,
      ],
  )
  def k(ei_hbm, ea_hbm, z_hbm, out_hbm,
        iv0, iv1, idx_x, rv0, rv1, acc, sem0, sem1, semi0, semi1, semx):
    c = lax.axis_index("c")
    s = lax.axis_index("s")
    wid = c * NS + s
    base = wid * E_PER_W
    xtra = (wid & 7) == 0
    t = wid >> 3

    pltpu.async_copy(ei_hbm.at[pl.ds(0, 2), pl.ds(base, CH)], iv0, semi0)
    pltpu.async_copy(ei_hbm.at[pl.ds(0, 2), pl.ds(base + CH, CH)], iv1, semi1)

    @pl.when(xtra)
    def _():
      pltpu.async_copy(ei_hbm.at[pl.ds(0, 2), pl.ds(X_BASE + t * CH, CH)],
                       idx_x, semx)

    pltpu.async_copy(ea_hbm.at[pl.ds(base, CH)], rv0, sem0)
    pltpu.async_copy(ea_hbm.at[pl.ds(base + CH, CH)], rv1, sem1)

    pltpu.sync_copy(z_hbm, acc.at[pl.ds(s * ROWS_PER_TILE, ROWS_PER_TILE)])

    @pl.when(s == NS - 1)
    def _():
      pltpu.sync_copy(z_hbm.at[pl.ds(0, TAIL_ROWS)],
                      acc.at[pl.ds(NS * ROWS_PER_TILE, TAIL_ROWS)])

    plsc.subcore_barrier()

    @pl.loop(0, NCH, step=2)
    def _(ch):
      for b in range(2):
        rv = (rv0, rv1)[b]
        iv = (iv0, iv1)[b]
        sem = (sem0, sem1)[b]
        semi = (semi0, semi1)[b]
        cc = ch + b
        pltpu.make_async_copy(ea_hbm.at[pl.ds(base, CH)], rv, sem).wait()
        pltpu.make_async_copy(ei_hbm.at[pl.ds(0, 2), pl.ds(base, CH)],
                              iv, semi).wait()
        pltpu.sync_copy(rv, acc.at[iv.at[1]], add=True)

        @pl.when(cc + 2 < NCH)
        def _():
          pltpu.async_copy(ea_hbm.at[pl.ds(base + (cc + 2) * CH, CH)], rv, sem)
          pltpu.async_copy(
              ei_hbm.at[pl.ds(0, 2), pl.ds(base + (cc + 2) * CH, CH)],
              iv, semi)

    @pl.when(xtra)
    def _():
      pltpu.sync_copy(ea_hbm.at[pl.ds(X_BASE + t * CH, CH)], rv0)
      pltpu.sync_copy(rv0, acc.at[idx_x.at[1]], add=True)

    plsc.subcore_barrier()
    r0 = s * ROWS_PER_TILE
    pltpu.sync_copy(acc.at[pl.ds(r0, ROWS_PER_TILE)],
                    out_hbm.at[c, pl.ds(r0, ROWS_PER_TILE)])

    @pl.when(s == NS - 1)
    def _():
      t0 = NS * ROWS_PER_TILE
      pltpu.sync_copy(acc.at[pl.ds(t0, TAIL_ROWS)],
                      out_hbm.at[c, pl.ds(t0, TAIL_ROWS)])

  return k(edge_index, edge_attr, zeros_tile)


def _mlp_body(x_ref, pa_ref, pb_ref, w1_ref, b1_ref, w2_ref,
              b2_ref, g_ref, bt_ref, o_ref):
  x = x_ref[...]
  agg = pa_ref[0] + pb_ref[0]
  xin = jnp.concatenate([x, agg], axis=-1)
  h = jnp.dot(xin, w1_ref[...], preferred_element_type=jnp.float32) + b1_ref[...]
  h = h * jax.nn.sigmoid(h)
  h = jnp.dot(h, w2_ref[...], preferred_element_type=jnp.float32) + b2_ref[...]
  mean = jnp.mean(h, axis=-1, keepdims=True)
  hc = h - mean
  var = jnp.mean(hc * hc, axis=-1, keepdims=True)
  o_ref[...] = x + hc * lax.rsqrt(var + 1e-5) * g_ref[...] + bt_ref[...]


def _tc_mlp(x, parts, W1, b1, W2, b2, gamma, beta):
  blk = 5000
  grid = N_NODES // blk
  row = lambda i: (i, 0)
  full = lambda i: (0, 0)
  return pl.pallas_call(
      _mlp_body,
      grid=(grid,),
      in_specs=[
          pl.BlockSpec((blk, D), row),
          pl.BlockSpec((1, blk, D), lambda i: (0, i, 0)),
          pl.BlockSpec((1, blk, D), lambda i: (1, i, 0)),
          pl.BlockSpec((256, 256), full),
          pl.BlockSpec((1, 256), full),
          pl.BlockSpec((256, D), full),
          pl.BlockSpec((1, D), full),
          pl.BlockSpec((1, D), full),
          pl.BlockSpec((1, D), full),
      ],
      out_specs=pl.BlockSpec((blk, D), row),
      out_shape=jax.ShapeDtypeStruct((N_NODES, D), jnp.float32),
  )(x, parts, parts, W1, b1, W2, b2, gamma, beta)


def kernel(x, edge_index, edge_attr, W1, b1, W2, b2, gamma, beta):
  ei = edge_index.astype(jnp.int32)
  zeros_tile = jnp.zeros((ROWS_PER_TILE, D), jnp.float32)
  parts = _sc_segment_sum(ei, edge_attr, zeros_tile)
  return _tc_mlp(x, parts, W1, b1.reshape(1, -1),
                 W2, b2.reshape(1, -1),
                 gamma.reshape(1, -1), beta.reshape(1, -1))


# speedup vs baseline: 8.3757x; 1.0002x over previous
import functools

import jax
import jax.numpy as jnp
from jax import lax
from jax.experimental import pallas as pl
from jax.experimental.pallas import tpu as pltpu
from jax.experimental.pallas import tpu_sc as plsc

N_NODES = 10000
N_EDGES = 320000
D = 128

NC = 2
NS = 16
NW = NC * NS
CH = 128
NCH = 78
E_PER_W = NCH * CH
X_BASE = NW * E_PER_W
ROWS_PER_TILE = 624
TAIL_ROWS = N_NODES - NS * ROWS_PER_TILE


def _sc_segment_sum(edge_index, edge_attr, zeros_tile):
  mesh = plsc.VectorSubcoreMesh(core_axis_name="c", subcore_axis_name="s")

  @functools.partial(
      pl.kernel,
      out_type=jax.ShapeDtypeStruct((NC, N_NODES, D), jnp.float32),
      mesh=mesh,
      scratch_types=[
          pltpu.VMEM((2, CH), jnp.int32),
          pltpu.VMEM((2, CH), jnp.int32),
          pltpu.VMEM((2, CH), jnp.int32),
          pltpu.VMEM((CH, D), jnp.float32),
          pltpu.VMEM((CH, D), jnp.float32),
          pltpu.VMEM_SHARED((N_NODES, D), jnp.float32),
          pltpu.SemaphoreType.DMA,
          pltpu.SemaphoreType.DMA,
          pltpu.SemaphoreType.DMA,
          pltpu.SemaphoreType.DMA,
          pltpu.SemaphoreType.DMA,
      ],
  )
  def k(ei_hbm, ea_hbm, z_hbm, out_hbm,
        iv0, iv1, idx_x, rv0, rv1, acc, sem0, sem1, semi0, semi1, semx):
    c = lax.axis_index("c")
    s = lax.axis_index("s")
    wid = c * NS + s
    base = wid * E_PER_W
    xtra = (wid & 7) == 0
    t = wid >> 3

    pltpu.async_copy(ei_hbm.at[pl.ds(0, 2), pl.ds(base, CH)], iv0, semi0)
    pltpu.async_copy(ei_hbm.at[pl.ds(0, 2), pl.ds(base + CH, CH)], iv1, semi1)

    @pl.when(xtra)
    def _():
      pltpu.async_copy(ei_hbm.at[pl.ds(0, 2), pl.ds(X_BASE + t * CH, CH)],
                       idx_x, semx)

    pltpu.async_copy(ea_hbm.at[pl.ds(base, CH)], rv0, sem0)
    pltpu.async_copy(ea_hbm.at[pl.ds(base + CH, CH)], rv1, sem1)

    pltpu.sync_copy(z_hbm, acc.at[pl.ds(s * ROWS_PER_TILE, ROWS_PER_TILE)])

    @pl.when(s == NS - 1)
    def _():
      pltpu.sync_copy(z_hbm.at[pl.ds(0, TAIL_ROWS)],
                      acc.at[pl.ds(NS * ROWS_PER_TILE, TAIL_ROWS)])

    plsc.subcore_barrier()

    @pl.loop(0, NCH, step=2)
    def _(ch):
      for b in range(2):
        rv = (rv0, rv1)[b]
        iv = (iv0, iv1)[b]
        sem = (sem0, sem1)[b]
        semi = (semi0, semi1)[b]
        cc = ch + b
        pltpu.make_async_copy(ea_hbm.at[pl.ds(base, CH)], rv, sem).wait()
        pltpu.make_async_copy(ei_hbm.at[pl.ds(0, 2), pl.ds(base, CH)],
                              iv, semi).wait()
        pltpu.sync_copy(rv, acc.at[iv.at[1]], add=True)

        @pl.when(cc + 2 < NCH)
        def _():
          pltpu.async_copy(ea_hbm.at[pl.ds(base + (cc + 2) * CH, CH)], rv, sem)
          pltpu.async_copy(
              ei_hbm.at[pl.ds(0, 2), pl.ds(base + (cc + 2) * CH, CH)],
              iv, semi)

    @pl.when(xtra)
    def _():
      pltpu.sync_copy(ea_hbm.at[pl.ds(X_BASE + t * CH, CH)], rv0)
      pltpu.sync_copy(rv0, acc.at[idx_x.at[1]], add=True)

    plsc.subcore_barrier()
    r0 = s * ROWS_PER_TILE
    pltpu.sync_copy(acc.at[pl.ds(r0, ROWS_PER_TILE)],
                    out_hbm.at[c, pl.ds(r0, ROWS_PER_TILE)])

    @pl.when(s == NS - 1)
    def _():
      t0 = NS * ROWS_PER_TILE
      pltpu.sync_copy(acc.at[pl.ds(t0, TAIL_ROWS)],
                      out_hbm.at[c, pl.ds(t0, TAIL_ROWS)])

  return k(edge_index, edge_attr, zeros_tile)


def _mlp_body(x_ref, pa_ref, pb_ref, w1_ref, b1_ref, w2_ref,
              b2_ref, g_ref, bt_ref, o_ref):
  x = x_ref[...]
  agg = pa_ref[0] + pb_ref[0]
  xin = jnp.concatenate([x, agg], axis=-1)
  h = jnp.dot(xin, w1_ref[...], preferred_element_type=jnp.float32) + b1_ref[...]
  h = h * jax.nn.sigmoid(h)
  h = jnp.dot(h, w2_ref[...], preferred_element_type=jnp.float32) + b2_ref[...]
  mean = jnp.mean(h, axis=-1, keepdims=True)
  hc = h - mean
  var = jnp.mean(hc * hc, axis=-1, keepdims=True)
  o_ref[...] = x + hc * lax.rsqrt(var + 1e-5) * g_ref[...] + bt_ref[...]


def _tc_mlp(x, parts, W1, b1, W2, b2, gamma, beta):
  blk = 5000
  grid = N_NODES // blk
  row = lambda i: (i, 0)
  full = lambda i: (0, 0)
  return pl.pallas_call(
      _mlp_body,
      grid=(grid,),
      in_specs=[
          pl.BlockSpec((blk, D), row),
          pl.BlockSpec((1, blk, D), lambda i: (0, i, 0)),
          pl.BlockSpec((1, blk, D), lambda i: (1, i, 0)),
          pl.BlockSpec((256, 256), full),
          pl.BlockSpec((1, 256), full),
          pl.BlockSpec((256, D), full),
          pl.BlockSpec((1, D), full),
          pl.BlockSpec((1, D), full),
          pl.BlockSpec((1, D), full),
      ],
      out_specs=pl.BlockSpec((blk, D), row),
      out_shape=jax.ShapeDtypeStruct((N_NODES, D), jnp.float32),
  )(x, parts, parts, W1, b1, W2, b2, gamma, beta)


def kernel(x, edge_index, edge_attr, W1, b1, W2, b2, gamma, beta):
  ei = edge_index.astype(jnp.int32)
  zeros_tile = jnp.zeros((ROWS_PER_TILE, D), jnp.float32)
  parts = _sc_segment_sum(ei, edge_attr, zeros_tile)
  return _tc_mlp(x, parts, W1, b1.reshape(1, -1),
                 W2, b2.reshape(1, -1),
                 gamma.reshape(1, -1), beta.reshape(1, -1))
